# Initial kernel scaffold; baseline (speedup 1.0000x reference)
import jax
import jax.numpy as jnp
from jax.experimental import pallas as pl


def kernel(x, token_emb, pos_emb):
    raise NotImplementedError("write your pallas kernel here")


# speedup vs baseline: 1.4915x; 1.4915x over previous
import functools

import jax
import jax.numpy as jnp
from jax import lax
from jax.experimental import pallas as pl
from jax.experimental.pallas import tpu as pltpu
from jax.experimental.pallas import tpu_sc as plsc

_B, _S, _D = 16, 2048, 64
_N = _B * _S
_NW = 32
_RPW = _N // _NW
_CHUNK = 512
_NCHUNK = _RPW // _CHUNK
_G = 128
_NG = _CHUNK // _G

_mesh = plsc.VectorSubcoreMesh(core_axis_name="c", subcore_axis_name="s")


@functools.partial(
    pl.kernel,
    mesh=_mesh,
    out_type=jax.ShapeDtypeStruct((_N, _D), jnp.float32),
    scratch_types=[
        pltpu.VMEM((_NCHUNK * _NG, _G), jnp.int32),
        pltpu.VMEM((_CHUNK, _D), jnp.float32),
        pltpu.VMEM((_CHUNK, _D), jnp.float32),
        pltpu.SemaphoreType.DMA,
    ],
    compiler_params=pltpu.CompilerParams(use_tc_tiling_on_sc=False),
)
def _emb_lookup(x_hbm, tok_hbm, pos_hbm, out_hbm, idx_v, rows_v, pos_v, sem):
    cid = lax.axis_index("c")
    sid = lax.axis_index("s")
    wid = sid * 2 + cid
    base = wid * _RPW
    pos_base = lax.rem(base, _S)

    pltpu.sync_copy(x_hbm.at[wid], idx_v)

    for k in range(_NCHUNK):
        row0 = base + k * _CHUNK
        cps = [
            pltpu.async_copy(
                tok_hbm.at[idx_v.at[k * _NG + g]],
                rows_v.at[pl.ds(g * _G, _G)],
                sem,
            )
            for g in range(_NG)
        ]
        pltpu.sync_copy(pos_hbm.at[pl.ds(pos_base + k * _CHUNK, _CHUNK)], pos_v)
        for cp in cps:
            cp.wait()

        def _add_row(r, carry):
            for c in range(_D // 16):
                sl = pl.ds(c * 16, 16)
                rows_v[r, sl] = rows_v[r, sl] + pos_v[r, sl]
            return carry

        lax.fori_loop(0, _CHUNK, _add_row, 0)

        pltpu.sync_copy(rows_v, out_hbm.at[pl.ds(row0, _CHUNK)])


def kernel(x, token_emb, pos_emb):
    idx = x.astype(jnp.int32).reshape(_NW, _NCHUNK * _NG, _G)
    out = _emb_lookup(idx, token_emb, pos_emb)
    return out.reshape(_B, _S, _D)


# speedup vs baseline: 1.4952x; 1.0025x over previous
import functools

import jax
import jax.numpy as jnp
from jax import lax
from jax.experimental import pallas as pl
from jax.experimental.pallas import tpu as pltpu
from jax.experimental.pallas import tpu_sc as plsc

_B, _S, _D = 16, 2048, 64
_N = _B * _S
_NW = 32
_RPW = _N // _NW
_CHUNK = 512
_NCHUNK = _RPW // _CHUNK
_G = 128
_NG = _CHUNK // _G

_mesh = plsc.VectorSubcoreMesh(core_axis_name="c", subcore_axis_name="s")


@functools.partial(
    pl.kernel,
    mesh=_mesh,
    out_type=jax.ShapeDtypeStruct((_N // 2, 128), jnp.float32),
    scratch_types=[
        pltpu.VMEM((_NCHUNK * _NG, _G), jnp.int32),
        pltpu.VMEM((_CHUNK, _D), jnp.float32),
        pltpu.VMEM((_CHUNK // 2, 128), jnp.float32),
        pltpu.SemaphoreType.DMA,
    ],
    compiler_params=pltpu.CompilerParams(use_tc_tiling_on_sc=False),
)
def _emb_lookup(x_hbm, tok_hbm, pos_hbm, out_hbm, idx_v, rows_v, pos_v, sem):
    cid = lax.axis_index("c")
    sid = lax.axis_index("s")
    wid = sid * 2 + cid
    base = wid * _RPW
    pos_base = lax.rem(base, _S)

    pltpu.sync_copy(x_hbm.at[pl.ds(wid * (_RPW // _G), _RPW // _G)], idx_v)

    for k in range(_NCHUNK):
        row0 = base + k * _CHUNK
        cps = [
            pltpu.async_copy(
                tok_hbm.at[idx_v.at[k * _NG + g]],
                rows_v.at[pl.ds(g * _G, _G)],
                sem,
            )
            for g in range(_NG)
        ]
        pltpu.sync_copy(
            pos_hbm.at[pl.ds((pos_base + k * _CHUNK) // 2, _CHUNK // 2)], pos_v
        )
        for cp in cps:
            cp.wait()

        def _add_row(r, carry):
            for c in range(8):
                pv = pl.ds(c * 16, 16)
                tv = pl.ds((c % 4) * 16, 16)
                pos_v[r, pv] = pos_v[r, pv] + rows_v[2 * r + c // 4, tv]
            return carry

        lax.fori_loop(0, _CHUNK // 2, _add_row, 0)

        pltpu.sync_copy(pos_v, out_hbm.at[pl.ds(row0 // 2, _CHUNK // 2)])


def kernel(x, token_emb, pos_emb):
    idx = x.astype(jnp.int32).reshape(_N // _G, _G)
    pos2 = pos_emb.reshape(_S // 2, 128)
    out = _emb_lookup(idx, token_emb, pos2)
    return out.reshape(_B, _S, _D)
